# baseline (speedup 1.0000x reference)
import jax
import jax.numpy as jnp
from jax.experimental import pallas as pl
from jax.experimental.pallas import tpu as pltpu

N = 256
K = 32
PACK = 4
NJ = N // PACK
PK = PACK * K
BPB = 4


def _swish_half(h):
    return h + h * jnp.tanh(h)


def _block_diag_mask(rows, row_blk):
    sub = jax.lax.broadcasted_iota(jnp.int32, (rows, PK), 0)
    lane = jax.lax.broadcasted_iota(jnp.int32, (rows, PK), 1)
    return (sub // row_blk) == (lane // K)


def _ogn_kernel(z4_ref, z64_ref, sp_ref, sp64_ref, wp_ref, out_ref):
    wp = wp_ref[...]
    w1 = 0.5 * wp[0:14]
    w1s, w1d = w1[0:6], w1[6:12]
    c1 = w1[12:13] + w1[13:14] + 0.5 * wp[14:15]
    c1t = jnp.tile(c1, (1, PACK))
    w1d_t = jnp.tile(w1d, (1, PACK))
    w1sq_t = jnp.tile(w1s[0:2], (1, PACK))
    w1dq_t = jnp.tile(w1d[0:2], (1, PACK))
    m8 = _block_diag_mask(2 * PACK, 2)
    q_bd = jnp.where(m8, jnp.tile(w1s[0:2], (PACK, PACK)), 0.0)
    p_bd = jnp.where(m8, jnp.tile(w1s[2:4], (PACK, PACK)), 0.0)
    s_bd = jnp.where(m8, jnp.tile(w1s[4:6], (PACK, PACK)), 0.0)
    w2bd = jnp.where(_block_diag_mask(PK, K),
                     jnp.tile(0.5 * wp[15:47], (PACK, PACK)),
                     0.0).astype(jnp.bfloat16)
    b2t = jnp.tile(0.5 * wp[47:48], (1, PACK))
    n1 = 0.5 * wp[48:87]
    nw1x = n1[0:6]
    nw1a4 = jnp.tile(n1[6:6 + K], (PACK, 1))
    nc1 = n1[6 + K:7 + K] + 0.5 * wp[87:88]
    nw2 = 0.5 * wp[88:120]
    nb2 = 0.5 * wp[120:121]

    for bb in range(BPB):
        q = z4_ref[bb, 0]
        p = z4_ref[bb, 1]
        sp = sp_ref[bb]
        x = jnp.concatenate([q, p, sp], axis=1)
        qm = jnp.mean(q, axis=0, keepdims=True)

        Aflat = (jnp.dot(z64_ref[bb, 0], q_bd, preferred_element_type=jnp.float32)
                 + jnp.dot(z64_ref[bb, 1], p_bd, preferred_element_type=jnp.float32)
                 + jnp.dot(sp64_ref[bb], s_bd, preferred_element_type=jnp.float32)
                 - jnp.dot(qm, w1sq_t, preferred_element_type=jnp.float32))
        Btile = (jnp.dot(x, w1d_t, preferred_element_type=jnp.float32)
                 - jnp.dot(qm, w1dq_t, preferred_element_type=jnp.float32)
                 + c1t)

        Afb = Aflat.astype(jnp.bfloat16)
        Btb = Btile.astype(jnp.bfloat16)
        Ab = jnp.broadcast_to(Afb[None], (N, NJ, PK)).reshape(N * NJ, PK)
        Bb = jnp.broadcast_to(Btb[:, None, :], (N, NJ, PK)).reshape(N * NJ, PK)

        preb = Ab + Bb
        h = preb + preb * jnp.tanh(preb)
        ep = _swish_half(jnp.dot(h, w2bd,
                                 preferred_element_type=jnp.float32) + b2t)

        s = ep.reshape(N, NJ, PK).sum(axis=1)

        nx = (jnp.dot(x, nw1x, preferred_element_type=jnp.float32)
              - jnp.dot(qm, nw1x[0:2], preferred_element_type=jnp.float32))
        h1 = _swish_half(nx + jnp.dot(s, nw1a4,
                                      preferred_element_type=jnp.float32) + nc1)
        vp = _swish_half(jnp.dot(h1, nw2,
                                 preferred_element_type=jnp.float32) + nb2)

        out_ref[bb, 0] = jnp.dot(vp, wp[121:153, 0:2],
                                 preferred_element_type=jnp.float32) + wp[153:154, 0:2]
        out_ref[bb, 1] = jnp.dot(vp, wp[154:186, 0:2],
                                 preferred_element_type=jnp.float32) + wp[186:187, 0:2]


def kernel(t, z, sysP, eW1, eb1, eW2, eb2, nW1, nb1, nW2, nb2,
           gW1, gb1, gW2, gb2, qW, qb, pW, pb):
    bs = z.shape[0]
    n = sysP.shape[1]
    d = z.shape[1] // (2 * n)
    sd = sysP.shape[2]
    z4 = z.reshape(bs, 2, n, d)
    z64 = z.reshape(bs, 2, NJ, PACK * d)
    sp64 = sysP.reshape(bs, NJ, PACK * sd)
    pad30 = lambda a: jnp.pad(a, ((0, 0), (0, K - a.shape[1])))
    wpack = jnp.concatenate([
        eW1, eb1[None], eW2, eb2[None], nW1, nb1[None], nW2, nb2[None],
        pad30(qW), pad30(qb[None]), pad30(pW), pad30(pb[None])], axis=0)

    wspec = lambda *shape: pl.BlockSpec(shape, lambda b: (0,) * len(shape))
    out = pl.pallas_call(
        _ogn_kernel,
        grid=(bs // BPB,),
        in_specs=[
            pl.BlockSpec((BPB, 2, n, d), lambda b: (b, 0, 0, 0)),
            pl.BlockSpec((BPB, 2, NJ, PACK * d), lambda b: (b, 0, 0, 0)),
            pl.BlockSpec((BPB, n, sd), lambda b: (b, 0, 0)),
            pl.BlockSpec((BPB, NJ, PACK * sd), lambda b: (b, 0, 0)),
            wspec(187, K),
        ],
        out_specs=pl.BlockSpec((BPB, 2, n, d), lambda b: (b, 0, 0, 0)),
        out_shape=jax.ShapeDtypeStruct((bs, 2, n, d), jnp.float32),
        compiler_params=pltpu.CompilerParams(
            dimension_semantics=("parallel",)),
    )(z4, z64, sysP, sp64, wpack)

    return out.reshape(bs, 2 * n * d)


# speedup vs baseline: 1.1108x; 1.1108x over previous
import jax
import jax.numpy as jnp
from jax.experimental import pallas as pl
from jax.experimental.pallas import tpu as pltpu

N = 256
K = 32
PACK = 4
NJ = N // PACK
PK = PACK * K
BPB = 4


def _swish_half(h):
    return h + h * jnp.tanh(h)


def _block_diag_mask(rows, row_blk):
    sub = jax.lax.broadcasted_iota(jnp.int32, (rows, PK), 0)
    lane = jax.lax.broadcasted_iota(jnp.int32, (rows, PK), 1)
    return (sub // row_blk) == (lane // K)


def _ogn_kernel(z4_ref, z64_ref, sp_ref, sp64_ref,
                ew1_ref, eb1_ref, ew2_ref, eb2_ref,
                nw1_ref, nb1_ref, nw2_ref, nb2_ref,
                qw_ref, qb_ref, pw_ref, pb_ref, out_ref):
    w1 = 0.5 * ew1_ref[...]
    w1s, w1d = w1[0:6], w1[6:12]
    c1 = w1[12:13] + w1[13:14] + 0.5 * eb1_ref[...]
    c1t = jnp.tile(c1, (1, PACK))
    w1d_t = jnp.tile(w1d, (1, PACK))
    w1sq_t = jnp.tile(w1s[0:2], (1, PACK))
    w1dq_t = jnp.tile(w1d[0:2], (1, PACK))
    m8 = _block_diag_mask(2 * PACK, 2)
    q_bd = jnp.where(m8, jnp.tile(w1s[0:2], (PACK, PACK)), 0.0)
    p_bd = jnp.where(m8, jnp.tile(w1s[2:4], (PACK, PACK)), 0.0)
    s_bd = jnp.where(m8, jnp.tile(w1s[4:6], (PACK, PACK)), 0.0)
    w2bd = jnp.where(_block_diag_mask(PK, K),
                     jnp.tile(0.5 * ew2_ref[...], (PACK, PACK)),
                     0.0).astype(jnp.bfloat16)
    b2t = jnp.tile(0.5 * eb2_ref[...], (1, PACK))
    n1 = 0.5 * nw1_ref[...]
    nw1x = n1[0:6]
    nw1a4 = jnp.tile(n1[6:6 + K], (PACK, 1))
    nc1 = n1[6 + K:7 + K] + 0.5 * nb1_ref[...]
    nw2 = 0.5 * nw2_ref[...]
    nb2 = 0.5 * nb2_ref[...]

    for bb in range(BPB):
        q = z4_ref[bb, 0]
        p = z4_ref[bb, 1]
        sp = sp_ref[bb]
        x = jnp.concatenate([q, p, sp], axis=1)
        qm = jnp.mean(q, axis=0, keepdims=True)

        Aflat = (jnp.dot(z64_ref[bb, 0], q_bd, preferred_element_type=jnp.float32)
                 + jnp.dot(z64_ref[bb, 1], p_bd, preferred_element_type=jnp.float32)
                 + jnp.dot(sp64_ref[bb], s_bd, preferred_element_type=jnp.float32)
                 - jnp.dot(qm, w1sq_t, preferred_element_type=jnp.float32))
        Btile = (jnp.dot(x, w1d_t, preferred_element_type=jnp.float32)
                 - jnp.dot(qm, w1dq_t, preferred_element_type=jnp.float32)
                 + c1t)

        Afb = Aflat.astype(jnp.bfloat16)
        Btb = Btile.astype(jnp.bfloat16)
        Ab = jnp.broadcast_to(Afb[None], (N, NJ, PK)).reshape(N * NJ, PK)
        Bb = jnp.broadcast_to(Btb[:, None, :], (N, NJ, PK)).reshape(N * NJ, PK)

        preb = Ab + Bb
        h = preb + preb * jnp.tanh(preb)
        ep = _swish_half(jnp.dot(h, w2bd,
                                 preferred_element_type=jnp.float32) + b2t)

        s = ep.reshape(N, NJ, PK).sum(axis=1)

        nx = (jnp.dot(x, nw1x, preferred_element_type=jnp.float32)
              - jnp.dot(qm, nw1x[0:2], preferred_element_type=jnp.float32))
        h1 = _swish_half(nx + jnp.dot(s, nw1a4,
                                      preferred_element_type=jnp.float32) + nc1)
        vp = _swish_half(jnp.dot(h1, nw2,
                                 preferred_element_type=jnp.float32) + nb2)

        out_ref[bb, 0] = jnp.dot(vp, qw_ref[...],
                                 preferred_element_type=jnp.float32) + qb_ref[...]
        out_ref[bb, 1] = jnp.dot(vp, pw_ref[...],
                                 preferred_element_type=jnp.float32) + pb_ref[...]


def kernel(t, z, sysP, eW1, eb1, eW2, eb2, nW1, nb1, nW2, nb2,
           gW1, gb1, gW2, gb2, qW, qb, pW, pb):
    bs = z.shape[0]
    n = sysP.shape[1]
    d = z.shape[1] // (2 * n)
    sd = sysP.shape[2]
    z4 = z.reshape(bs, 2, n, d)
    z64 = z.reshape(bs, 2, NJ, PACK * d)
    sp64 = sysP.reshape(bs, NJ, PACK * sd)

    wspec = lambda *shape: pl.BlockSpec(shape, lambda b: (0,) * len(shape))
    out = pl.pallas_call(
        _ogn_kernel,
        grid=(bs // BPB,),
        in_specs=[
            pl.BlockSpec((BPB, 2, n, d), lambda b: (b, 0, 0, 0)),
            pl.BlockSpec((BPB, 2, NJ, PACK * d), lambda b: (b, 0, 0, 0)),
            pl.BlockSpec((BPB, n, sd), lambda b: (b, 0, 0)),
            pl.BlockSpec((BPB, NJ, PACK * sd), lambda b: (b, 0, 0)),
            wspec(14, K), wspec(1, K),
            wspec(K, K), wspec(1, K),
            wspec(39, K), wspec(1, K),
            wspec(K, K), wspec(1, K),
            wspec(K, d), wspec(1, d),
            wspec(K, d), wspec(1, d),
        ],
        out_specs=pl.BlockSpec((BPB, 2, n, d), lambda b: (b, 0, 0, 0)),
        out_shape=jax.ShapeDtypeStruct((bs, 2, n, d), jnp.float32),
        compiler_params=pltpu.CompilerParams(
            dimension_semantics=("parallel",)),
    )(z4, z64, sysP, sp64,
      eW1, eb1.reshape(1, K), eW2, eb2.reshape(1, K),
      nW1, nb1.reshape(1, K), nW2, nb2.reshape(1, K),
      qW, qb.reshape(1, d), pW, pb.reshape(1, d))

    return out.reshape(bs, 2 * n * d)
